# Initial kernel scaffold; baseline (speedup 1.0000x reference)
import jax
import jax.numpy as jnp
from jax.experimental import pallas as pl


def kernel(x, edge_attr, edge_index, Wn1, bn1, We1, be1, Wc1, bc1, Wn2, bn2, We2, be2, Wc2, bc2, Wn3, bn3, We3, be3, Wc3, bc3, Wg, a_src, a_dst, bg):
    raise NotImplementedError("write your pallas kernel here")


# speedup vs baseline: 10.9902x; 10.9902x over previous
import functools

import jax
import jax.numpy as jnp
from jax.experimental import pallas as pl

_N = 10000
_NP = 10240
_BD = 512
_BS = 512
_ND = _NP // _BD
_NS = _NP // _BS
_NEG = -jnp.inf


def _lin_kernel(x_ref, w_ref, b_ref, o_ref):
    o_ref[...] = (
        jnp.dot(x_ref[...], w_ref[...], preferred_element_type=jnp.float32)
        + b_ref[...]
    )


def _lin(x, w, b, bn=2048):
    n = x.shape[0]
    k = x.shape[1]
    dout = w.shape[1]
    return pl.pallas_call(
        _lin_kernel,
        grid=(n // bn,),
        in_specs=[
            pl.BlockSpec((bn, k), lambda i: (i, 0)),
            pl.BlockSpec((k, dout), lambda i: (0, 0)),
            pl.BlockSpec((1, dout), lambda i: (0, 0)),
        ],
        out_specs=pl.BlockSpec((bn, dout), lambda i: (i, 0)),
        out_shape=jax.ShapeDtypeStruct((n, dout), jnp.float32),
    )(x, w, b.reshape(1, dout))


def _hp_al_kernel(x_ref, w_ref, c_ref, hp_ref, al_ref):
    hp = jnp.dot(x_ref[...], w_ref[...], preferred_element_type=jnp.float32)
    hp_ref[...] = hp
    al_ref[...] = jnp.dot(hp, c_ref[...], preferred_element_type=jnp.float32)


def _hp_al(h3, wg, c, bn=2048):
    return pl.pallas_call(
        _hp_al_kernel,
        grid=(_NP // bn,),
        in_specs=[
            pl.BlockSpec((bn, 64), lambda i: (i, 0)),
            pl.BlockSpec((64, 128), lambda i: (0, 0)),
            pl.BlockSpec((128, 8), lambda i: (0, 0)),
        ],
        out_specs=(
            pl.BlockSpec((bn, 128), lambda i: (i, 0)),
            pl.BlockSpec((bn, 8), lambda i: (i, 0)),
        ),
        out_shape=(
            jax.ShapeDtypeStruct((_NP, 128), jnp.float32),
            jax.ShapeDtypeStruct((_NP, 8), jnp.float32),
        ),
    )(h3, wg, c)


def _conv_kernel(a_ref, hn_ref, g_ref, wch_ref, wl_ref, o_ref):
    s = pl.program_id(1)
    ns = pl.num_programs(1)
    part = jnp.dot(a_ref[...], hn_ref[...], preferred_element_type=jnp.float32)

    @pl.when(s == 0)
    def _():
        o_ref[...] = part

    @pl.when(s > 0)
    def _():
        o_ref[...] += part

    @pl.when(s == ns - 1)
    def _():
        z = o_ref[...]
        o_ref[...] = jnp.maximum(
            jnp.dot(z, wch_ref[...], preferred_element_type=jnp.float32)
            + jnp.dot(g_ref[...], wl_ref[...], preferred_element_type=jnp.float32),
            0.0,
        )


def _conv(a, hn, g, wch, wl8):
    return pl.pallas_call(
        _conv_kernel,
        grid=(_ND, _NS),
        in_specs=[
            pl.BlockSpec((_BD, _BS), lambda d, s: (d, s)),
            pl.BlockSpec((_BS, 64), lambda d, s: (s, 0)),
            pl.BlockSpec((_BD, 8), lambda d, s: (d, 0)),
            pl.BlockSpec((64, 64), lambda d, s: (0, 0)),
            pl.BlockSpec((8, 64), lambda d, s: (0, 0)),
        ],
        out_specs=pl.BlockSpec((_BD, 64), lambda d, s: (d, 0)),
        out_shape=jax.ShapeDtypeStruct((_NP, 64), jnp.float32),
    )(a, hn, g, wch, wl8)


def _max_kernel(a_ref, alst_ref, o_ref):
    s = pl.program_id(1)
    a = a_ref[...]
    cols = []
    for k in range(2):
        masked = jnp.where(a > 0.0, alst_ref[k : k + 1, :], _NEG)
        cols.append(jnp.max(masked, axis=1)[:, None])
    cols.append(jnp.full((a.shape[0], 6), _NEG, jnp.float32))
    cur = jnp.concatenate(cols, axis=1)

    @pl.when(s == 0)
    def _():
        o_ref[...] = cur

    @pl.when(s > 0)
    def _():
        o_ref[...] = jnp.maximum(o_ref[...], cur)


def _max_pass(a, alst):
    return pl.pallas_call(
        _max_kernel,
        grid=(_ND, _NS),
        in_specs=[
            pl.BlockSpec((_BD, _BS), lambda d, s: (d, s)),
            pl.BlockSpec((8, _BS), lambda d, s: (0, s)),
        ],
        out_specs=pl.BlockSpec((_BD, 8), lambda d, s: (d, 0)),
        out_shape=jax.ShapeDtypeStruct((_NP, 8), jnp.float32),
    )(a, alst)


def _gat_kernel(a_ref, alst_ref, al_ref, m_ref, hp_ref, num_ref, den_ref):
    s = pl.program_id(1)
    a = a_ref[...]
    nums = []
    dens = []
    for k in range(2):
        ald = al_ref[:, 2 + k : 3 + k]
        z = ald + alst_ref[k : k + 1, :]
        lr = jnp.maximum(z, 0.2 * z)
        mm = m_ref[:, k : k + 1] + ald
        m = jnp.maximum(mm, 0.2 * mm)
        e = jnp.exp(lr - m)
        b = jnp.where(a > 0.0, a * e, 0.0)
        nums.append(
            jnp.dot(b, hp_ref[:, 64 * k : 64 * (k + 1)],
                    preferred_element_type=jnp.float32)
        )
        dens.append(jnp.sum(b, axis=1)[:, None])
    dens.append(jnp.zeros((a.shape[0], 6), jnp.float32))
    num = jnp.concatenate(nums, axis=1)
    den = jnp.concatenate(dens, axis=1)

    @pl.when(s == 0)
    def _():
        num_ref[...] = num
        den_ref[...] = den

    @pl.when(s > 0)
    def _():
        num_ref[...] += num
        den_ref[...] += den


def _gat_pass(a, alst, al, m, hp):
    return pl.pallas_call(
        _gat_kernel,
        grid=(_ND, _NS),
        in_specs=[
            pl.BlockSpec((_BD, _BS), lambda d, s: (d, s)),
            pl.BlockSpec((8, _BS), lambda d, s: (0, s)),
            pl.BlockSpec((_BD, 8), lambda d, s: (d, 0)),
            pl.BlockSpec((_BD, 8), lambda d, s: (d, 0)),
            pl.BlockSpec((_BS, 128), lambda d, s: (s, 0)),
        ],
        out_specs=(
            pl.BlockSpec((_BD, 128), lambda d, s: (d, 0)),
            pl.BlockSpec((_BD, 8), lambda d, s: (d, 0)),
        ),
        out_shape=(
            jax.ShapeDtypeStruct((_NP, 128), jnp.float32),
            jax.ShapeDtypeStruct((_NP, 8), jnp.float32),
        ),
    )(a, alst, al, m, hp)


@jax.jit
def _impl(x, edge_attr, edge_index, Wn1, bn1, We1, be1, Wc1, bc1, Wn2, bn2,
          We2, be2, Wc2, bc2, Wn3, bn3, We3, be3, Wc3, bc3, Wg, a_src,
          a_dst, bg):
    src = edge_index[0].astype(jnp.int32)
    dst = edge_index[1].astype(jnp.int32)
    e = src.shape[0]

    a = jnp.zeros((_NP, _NP), jnp.float32).at[dst, src].add(1.0)
    gfeat = jnp.concatenate(
        [edge_attr, jnp.ones((e, 1), jnp.float32),
         jnp.zeros((e, 2), jnp.float32)], axis=1)
    g = jnp.zeros((_NP, 8), jnp.float32).at[dst].add(gfeat)

    def combo(We, be, Wc, bc):
        wch = Wc[:64]
        wce = Wc[64:]
        wl = jnp.concatenate(
            [We @ wce, (be @ wce + bc)[None], jnp.zeros((2, 64), jnp.float32)],
            axis=0)
        return wch, wl

    wch1, wl1 = combo(We1, be1, Wc1, bc1)
    wch2, wl2 = combo(We2, be2, Wc2, bc2)
    wch3, wl3 = combo(We3, be3, Wc3, bc3)

    xp = jnp.pad(x, ((0, _NP - _N), (0, 0)))
    hn = _lin(xp, Wn1, bn1)
    h = _conv(a, hn, g, wch1, wl1)
    hn = _lin(h, Wn2, bn2)
    h = _conv(a, hn, g, wch2, wl2)
    hn = _lin(h, Wn3, bn3)
    h = _conv(a, hn, g, wch3, wl3)

    c = jnp.zeros((128, 8), jnp.float32)
    c = c.at[:64, 0].set(a_src[0]).at[64:, 1].set(a_src[1])
    c = c.at[:64, 2].set(a_dst[0]).at[64:, 3].set(a_dst[1])
    hp, al = _hp_al(h, Wg, c)

    alst = jnp.pad(al[:, :2].T, ((0, 6), (0, 0)))
    m = _max_pass(a, alst)
    num, den = _gat_pass(a, alst, al, m, hp)

    out0 = num[:_N, :64] / (den[:_N, 0:1] + 1e-16)
    out1 = num[:_N, 64:128] / (den[:_N, 1:2] + 1e-16)
    out = 0.5 * (out0 + out1) + bg
    return out.reshape(-1)


def kernel(x, edge_attr, edge_index, Wn1, bn1, We1, be1, Wc1, bc1, Wn2, bn2,
           We2, be2, Wc2, bc2, Wn3, bn3, We3, be3, Wc3, bc3, Wg, a_src,
           a_dst, bg):
    return _impl(x, edge_attr, edge_index, Wn1, bn1, We1, be1, Wc1, bc1,
                 Wn2, bn2, We2, be2, Wc2, bc2, Wn3, bn3, We3, be3, Wc3, bc3,
                 Wg, a_src, a_dst, bg)
